# baseline (speedup 1.0000x reference)
import functools

import jax
import jax.numpy as jnp
from jax.experimental import pallas as pl
from jax.experimental.pallas import tpu as pltpu
from jax.experimental.pallas import tpu_sc as plsc

B, S = 8, 64
V, D, U = 2048, 22000, 64
BS = B * S
G4 = 4 * U
KT = 1024
NK = (D + KT - 1) // KT


def _mproj_body(et_ref, wi_ref, b_ref, m_ref):
    j = pl.program_id(0)

    @pl.when(j == 0)
    def _():
        m_ref[...] = jnp.broadcast_to(b_ref[...], (V, G4))

    et = et_ref[...]
    wi = wi_ref[...]

    def operands_plain():
        return et, wi

    def operands_masked():
        row = j * KT + jax.lax.broadcasted_iota(jnp.int32, (KT, 1), 0)
        return (jnp.where(row < D, et, 0.0), jnp.where(row < D, wi, 0.0))

    et, wi = jax.lax.cond(j == NK - 1, operands_masked, operands_plain)
    m_ref[...] += jax.lax.dot_general(
        et.astype(jnp.bfloat16), wi.astype(jnp.bfloat16),
        dimension_numbers=(((0,), (0,)), ((), ())),
        preferred_element_type=jnp.float32)


def _mproj(emb_t, Wi, b):
    return pl.pallas_call(
        _mproj_body,
        grid=(NK,),
        in_specs=[
            pl.BlockSpec((KT, V), lambda j: (j, 0)),
            pl.BlockSpec((KT, G4), lambda j: (j, 0)),
            pl.BlockSpec((1, G4), lambda j: (0, 0)),
        ],
        out_specs=pl.BlockSpec((V, G4), lambda j: (0, 0)),
        out_shape=jax.ShapeDtypeStruct((V, G4), jnp.float32),
        compiler_params=pltpu.CompilerParams(
            fuse_transposed_lhs_in_matmul=True),
    )(emb_t, Wi, b.reshape(1, G4))


def _sc_gather_rows(m, idx):
    info = plsc.get_sparse_core_info()
    nw = info.num_cores * info.num_subcores
    bpw = BS // nw
    mesh = plsc.VectorSubcoreMesh(core_axis_name="c", subcore_axis_name="s")

    @functools.partial(
        pl.kernel,
        mesh=mesh,
        out_type=jax.ShapeDtypeStruct((BS, G4), jnp.float32),
        scratch_types=[
            pltpu.VMEM((bpw,), jnp.int32),
            pltpu.VMEM((bpw, G4), jnp.float32),
            pltpu.SemaphoreType.DMA,
        ],
    )
    def k(m_hbm, idx_hbm, out_hbm, idx_v, rows_v, sem):
        wid = jax.lax.axis_index("s") * info.num_cores + jax.lax.axis_index("c")
        base = wid * bpw
        pltpu.sync_copy(idx_hbm.at[pl.ds(base, bpw)], idx_v)
        pltpu.async_copy(m_hbm.at[idx_v], rows_v, sem).wait()
        pltpu.sync_copy(rows_v, out_hbm.at[pl.ds(base, bpw)])

    return k(m, idx)


def _gates(z, c):
    i = jax.nn.sigmoid(z[:, 0 * U:1 * U])
    f = jax.nn.sigmoid(z[:, 1 * U:2 * U])
    g = jnp.tanh(z[:, 2 * U:3 * U])
    o = jax.nn.sigmoid(z[:, 3 * U:4 * U])
    c = f * c + i * g
    h = o * jnp.tanh(c)
    return h, c


def _rec_body(ze_ref, zd_ref, whe_ref, whd_ref, out_ref):
    whe = whe_ref[...]
    whd = whd_ref[...]

    def enc_step(t, carry):
        h, c = carry
        z = ze_ref[t] + jnp.dot(h, whe, preferred_element_type=jnp.float32)
        return _gates(z, c)

    zero = jnp.zeros((B, U), jnp.float32)
    h_e, c_e = jax.lax.fori_loop(0, S, enc_step, (zero, zero))

    def dec_step(t, carry):
        h, c = carry
        z = zd_ref[t] + jnp.dot(h, whd, preferred_element_type=jnp.float32)
        h, c = _gates(z, c)
        out_ref[t] = h
        return (h, c)

    jax.lax.fori_loop(0, S, dec_step, (h_e, c_e))


def _recurrence(z_e_t, z_d_t, Wh_e, Wh_d):
    return pl.pallas_call(
        _rec_body,
        out_shape=jax.ShapeDtypeStruct((S, B, U), jnp.float32),
    )(z_e_t, z_d_t, Wh_e, Wh_d)


def _head_body(x_ref, wd_ref, bd_ref, o_ref):
    logits = (
        jnp.dot(x_ref[...], wd_ref[...], preferred_element_type=jnp.float32)
        + bd_ref[...]
    )
    m = jnp.max(logits, axis=1, keepdims=True)
    e = jnp.exp(logits - m)
    o_ref[...] = e / jnp.sum(e, axis=1, keepdims=True)


def _softmax_head(x, Wd, bd):
    return pl.pallas_call(
        _head_body,
        out_shape=jax.ShapeDtypeStruct((BS, V), jnp.float32),
    )(x, Wd, bd.reshape(1, V))


def kernel(encoder_input, decoder_input, emb1, emb2, Wi_e, Wh_e, b_e,
           Wi_d, Wh_d, b_d, Wd, bd):
    idx_e = encoder_input.reshape(BS)
    idx_d = decoder_input.reshape(BS)
    m_e = _mproj(emb1.T, Wi_e, b_e)
    m_d = _mproj(emb2.T, Wi_d, b_d)
    z_e = _sc_gather_rows(m_e, idx_e)
    z_d = _sc_gather_rows(m_d, idx_d)
    z_e_t = z_e.reshape(B, S, G4).transpose(1, 0, 2)
    z_d_t = z_d.reshape(B, S, G4).transpose(1, 0, 2)
    dec_out = _recurrence(z_e_t, z_d_t, Wh_e, Wh_d)
    x = dec_out.transpose(1, 0, 2).reshape(BS, U)
    prbs = _softmax_head(x, Wd, bd)
    return prbs.reshape(B, S, V)


# speedup vs baseline: 1.1055x; 1.1055x over previous
import functools

import jax
import jax.numpy as jnp
from jax.experimental import pallas as pl
from jax.experimental.pallas import tpu as pltpu
from jax.experimental.pallas import tpu_sc as plsc

B, S = 8, 64
V, D, U = 2048, 22000, 64
BS = B * S
G4 = 4 * U
KT = 1024
NK = (D + KT - 1) // KT


def _mproj_body(et_ref, wi_ref, b_ref, mt_ref):
    j = pl.program_id(0)

    @pl.when(j == 0)
    def _():
        mt_ref[...] = jnp.broadcast_to(b_ref[...], (G4, V))

    et = et_ref[...]
    wi = wi_ref[...]

    def operands_plain():
        return et, wi

    def operands_masked():
        row = j * KT + jax.lax.broadcasted_iota(jnp.int32, (KT, 1), 0)
        return (jnp.where(row < D, et, 0.0), jnp.where(row < D, wi, 0.0))

    et, wi = jax.lax.cond(j == NK - 1, operands_masked, operands_plain)
    wi_t = wi.astype(jnp.bfloat16).T
    mt_ref[...] += jnp.dot(
        wi_t, et.astype(jnp.bfloat16), preferred_element_type=jnp.float32)


def _mproj(emb_t, Wi, b):
    return pl.pallas_call(
        _mproj_body,
        grid=(NK,),
        in_specs=[
            pl.BlockSpec((KT, V), lambda j: (j, 0)),
            pl.BlockSpec((KT, G4), lambda j: (j, 0)),
            pl.BlockSpec((G4, 1), lambda j: (0, 0)),
        ],
        out_specs=pl.BlockSpec((G4, V), lambda j: (0, 0)),
        out_shape=jax.ShapeDtypeStruct((G4, V), jnp.float32),
    )(emb_t, Wi, b.reshape(G4, 1))


def _sc_gather_rows(m, idx):
    info = plsc.get_sparse_core_info()
    nw = info.num_cores * info.num_subcores
    bpw = BS // nw
    mesh = plsc.VectorSubcoreMesh(core_axis_name="c", subcore_axis_name="s")

    @functools.partial(
        pl.kernel,
        mesh=mesh,
        out_type=jax.ShapeDtypeStruct((BS, G4), jnp.float32),
        scratch_types=[
            pltpu.VMEM((bpw,), jnp.int32),
            pltpu.VMEM((bpw, G4), jnp.float32),
            pltpu.SemaphoreType.DMA,
        ],
    )
    def k(m_hbm, idx_hbm, out_hbm, idx_v, rows_v, sem):
        wid = jax.lax.axis_index("s") * info.num_cores + jax.lax.axis_index("c")
        base = wid * bpw
        pltpu.sync_copy(idx_hbm.at[pl.ds(base, bpw)], idx_v)
        pltpu.async_copy(m_hbm.at[idx_v], rows_v, sem).wait()
        pltpu.sync_copy(rows_v, out_hbm.at[pl.ds(base, bpw)])

    return k(m, idx)


def _gates(z, c):
    i = jax.nn.sigmoid(z[:, 0 * U:1 * U])
    f = jax.nn.sigmoid(z[:, 1 * U:2 * U])
    g = jnp.tanh(z[:, 2 * U:3 * U])
    o = jax.nn.sigmoid(z[:, 3 * U:4 * U])
    c = f * c + i * g
    h = o * jnp.tanh(c)
    return h, c


def _rec_body(ze_ref, zd_ref, whe_ref, whd_ref, out_ref):
    whe = whe_ref[...]
    whd = whd_ref[...]

    def enc_step(t, carry):
        h, c = carry
        z = ze_ref[t] + jnp.dot(h, whe, preferred_element_type=jnp.float32)
        return _gates(z, c)

    zero = jnp.zeros((B, U), jnp.float32)
    h_e, c_e = jax.lax.fori_loop(0, S, enc_step, (zero, zero))

    def dec_step(t, carry):
        h, c = carry
        z = zd_ref[t] + jnp.dot(h, whd, preferred_element_type=jnp.float32)
        h, c = _gates(z, c)
        out_ref[t] = h
        return (h, c)

    jax.lax.fori_loop(0, S, dec_step, (h_e, c_e))


def _recurrence(z_e_t, z_d_t, Wh_e, Wh_d):
    return pl.pallas_call(
        _rec_body,
        out_shape=jax.ShapeDtypeStruct((S, B, U), jnp.float32),
    )(z_e_t, z_d_t, Wh_e, Wh_d)


def _head_body(x_ref, wd_ref, bd_ref, o_ref):
    logits = (
        jnp.dot(x_ref[...], wd_ref[...], preferred_element_type=jnp.float32)
        + bd_ref[...]
    )
    m = jnp.max(logits, axis=1, keepdims=True)
    e = jnp.exp(logits - m)
    o_ref[...] = e / jnp.sum(e, axis=1, keepdims=True)


def _softmax_head(x, Wd, bd):
    return pl.pallas_call(
        _head_body,
        out_shape=jax.ShapeDtypeStruct((BS, V), jnp.float32),
    )(x, Wd, bd.reshape(1, V))


def kernel(encoder_input, decoder_input, emb1, emb2, Wi_e, Wh_e, b_e,
           Wi_d, Wh_d, b_d, Wd, bd):
    idx_e = encoder_input.reshape(BS)
    idx_d = decoder_input.reshape(BS)
    m_e = _mproj(emb1.T, Wi_e, b_e).T
    m_d = _mproj(emb2.T, Wi_d, b_d).T
    z_e = _sc_gather_rows(m_e, idx_e)
    z_d = _sc_gather_rows(m_d, idx_d)
    z_e_t = z_e.reshape(B, S, G4).transpose(1, 0, 2)
    z_d_t = z_d.reshape(B, S, G4).transpose(1, 0, 2)
    dec_out = _recurrence(z_e_t, z_d_t, Wh_e, Wh_d)
    x = dec_out.transpose(1, 0, 2).reshape(BS, U)
    prbs = _softmax_head(x, Wd, bd)
    return prbs.reshape(B, S, V)


# speedup vs baseline: 1.6957x; 1.5339x over previous
import functools

import jax
import jax.numpy as jnp
from jax.experimental import pallas as pl
from jax.experimental.pallas import tpu as pltpu
from jax.experimental.pallas import tpu_sc as plsc

B, S = 8, 64
V, D, U = 2048, 22000, 64
BS = B * S
G4 = 4 * U
KT = 1024
NK = (D + KT - 1) // KT


def _mproj_body(et_ref, wi_ref, b_ref, m_ref):
    j = pl.program_id(0)

    @pl.when(j == 0)
    def _():
        m_ref[...] = jnp.broadcast_to(b_ref[...], (V, G4))

    row = j * KT + jax.lax.broadcasted_iota(jnp.int32, (KT, 1), 0)
    et = jnp.where(row < D, et_ref[...], 0.0)
    wi = jnp.where(row < D, wi_ref[...], 0.0)
    m_ref[...] += jax.lax.dot_general(
        et.astype(jnp.bfloat16), wi.astype(jnp.bfloat16),
        dimension_numbers=(((0,), (0,)), ((), ())),
        preferred_element_type=jnp.float32)


def _mproj(emb_t, Wi, b):
    return pl.pallas_call(
        _mproj_body,
        grid=(NK,),
        in_specs=[
            pl.BlockSpec((KT, V), lambda j: (j, 0)),
            pl.BlockSpec((KT, G4), lambda j: (j, 0)),
            pl.BlockSpec((1, G4), lambda j: (0, 0)),
        ],
        out_specs=pl.BlockSpec((V, G4), lambda j: (0, 0)),
        out_shape=jax.ShapeDtypeStruct((V, G4), jnp.float32),
    )(emb_t, Wi, b.reshape(1, G4))


def _sc_gather_rows(m, idx):
    info = plsc.get_sparse_core_info()
    nw = info.num_cores * info.num_subcores
    bpw = BS // nw
    mesh = plsc.VectorSubcoreMesh(core_axis_name="c", subcore_axis_name="s")

    @functools.partial(
        pl.kernel,
        mesh=mesh,
        out_type=jax.ShapeDtypeStruct((BS, G4), jnp.float32),
        scratch_types=[
            pltpu.VMEM((bpw,), jnp.int32),
            pltpu.VMEM((bpw, G4), jnp.float32),
            pltpu.SemaphoreType.DMA,
        ],
    )
    def k(m_hbm, idx_hbm, out_hbm, idx_v, rows_v, sem):
        wid = jax.lax.axis_index("s") * info.num_cores + jax.lax.axis_index("c")
        base = wid * bpw
        pltpu.sync_copy(idx_hbm.at[pl.ds(base, bpw)], idx_v)
        pltpu.async_copy(m_hbm.at[idx_v], rows_v, sem).wait()
        pltpu.sync_copy(rows_v, out_hbm.at[pl.ds(base, bpw)])

    return k(m, idx)


def _gates(z, c):
    i = jax.nn.sigmoid(z[:, 0 * U:1 * U])
    f = jax.nn.sigmoid(z[:, 1 * U:2 * U])
    g = jnp.tanh(z[:, 2 * U:3 * U])
    o = jax.nn.sigmoid(z[:, 3 * U:4 * U])
    c = f * c + i * g
    h = o * jnp.tanh(c)
    return h, c


def _rec_body(ze_ref, zd_ref, whe_ref, whd_ref, out_ref):
    whe = whe_ref[...]
    whd = whd_ref[...]

    def enc_step(t, carry):
        h, c = carry
        z = ze_ref[t] + jnp.dot(h, whe, preferred_element_type=jnp.float32)
        return _gates(z, c)

    zero = jnp.zeros((B, U), jnp.float32)
    h_e, c_e = jax.lax.fori_loop(0, S, enc_step, (zero, zero))

    def dec_step(t, carry):
        h, c = carry
        z = zd_ref[t] + jnp.dot(h, whd, preferred_element_type=jnp.float32)
        h, c = _gates(z, c)
        out_ref[t] = h
        return (h, c)

    jax.lax.fori_loop(0, S, dec_step, (h_e, c_e))


def _recurrence(z_e_t, z_d_t, Wh_e, Wh_d):
    return pl.pallas_call(
        _rec_body,
        out_shape=jax.ShapeDtypeStruct((S, B, U), jnp.float32),
    )(z_e_t, z_d_t, Wh_e, Wh_d)


def _head_body(x_ref, wd_ref, bd_ref, o_ref):
    logits = (
        jnp.dot(x_ref[...], wd_ref[...], preferred_element_type=jnp.float32)
        + bd_ref[...]
    )
    m = jnp.max(logits, axis=1, keepdims=True)
    e = jnp.exp(logits - m)
    o_ref[...] = e / jnp.sum(e, axis=1, keepdims=True)


def _softmax_head(x, Wd, bd):
    return pl.pallas_call(
        _head_body,
        out_shape=jax.ShapeDtypeStruct((BS, V), jnp.float32),
    )(x, Wd, bd.reshape(1, V))


def kernel(encoder_input, decoder_input, emb1, emb2, Wi_e, Wh_e, b_e,
           Wi_d, Wh_d, b_d, Wd, bd):
    idx_e = encoder_input.reshape(BS)
    idx_d = decoder_input.reshape(BS)
    m_e = _mproj(emb1.T, Wi_e, b_e)
    m_d = _mproj(emb2.T, Wi_d, b_d)
    z_e = _sc_gather_rows(m_e, idx_e)
    z_d = _sc_gather_rows(m_d, idx_d)
    z_e_t = z_e.reshape(B, S, G4).transpose(1, 0, 2)
    z_d_t = z_d.reshape(B, S, G4).transpose(1, 0, 2)
    dec_out = _recurrence(z_e_t, z_d_t, Wh_e, Wh_d)
    x = dec_out.transpose(1, 0, 2).reshape(BS, U)
    prbs = _softmax_head(x, Wd, bd)
    return prbs.reshape(B, S, V)


# speedup vs baseline: 1.7221x; 1.0155x over previous
import functools

import jax
import jax.numpy as jnp
from jax.experimental import pallas as pl
from jax.experimental.pallas import tpu as pltpu
from jax.experimental.pallas import tpu_sc as plsc

B, S = 8, 64
V, D, U = 2048, 22000, 64
BS = B * S
G4 = 4 * U
KT = 2048
NK = (D + KT - 1) // KT


def _mproj_body(et_ref, wi_ref, b_ref, m_ref):
    j = pl.program_id(0)

    @pl.when(j == 0)
    def _():
        m_ref[...] = jnp.broadcast_to(b_ref[...], (V, G4))

    row = j * KT + jax.lax.broadcasted_iota(jnp.int32, (KT, 1), 0)
    et = jnp.where(row < D, et_ref[...], 0.0)
    wi = jnp.where(row < D, wi_ref[...], 0.0)
    m_ref[...] += jax.lax.dot_general(
        et.astype(jnp.bfloat16), wi.astype(jnp.bfloat16),
        dimension_numbers=(((0,), (0,)), ((), ())),
        preferred_element_type=jnp.float32)


def _mproj(emb_t, Wi, b):
    return pl.pallas_call(
        _mproj_body,
        grid=(NK,),
        in_specs=[
            pl.BlockSpec((KT, V), lambda j: (j, 0)),
            pl.BlockSpec((KT, G4), lambda j: (j, 0)),
            pl.BlockSpec((1, G4), lambda j: (0, 0)),
        ],
        out_specs=pl.BlockSpec((V, G4), lambda j: (0, 0)),
        out_shape=jax.ShapeDtypeStruct((V, G4), jnp.float32),
    )(emb_t, Wi, b.reshape(1, G4))


def _sc_gather_rows(m, idx):
    info = plsc.get_sparse_core_info()
    nw = info.num_cores * info.num_subcores
    bpw = BS // nw
    mesh = plsc.VectorSubcoreMesh(core_axis_name="c", subcore_axis_name="s")

    @functools.partial(
        pl.kernel,
        mesh=mesh,
        out_type=jax.ShapeDtypeStruct((BS, G4), jnp.float32),
        scratch_types=[
            pltpu.VMEM((bpw,), jnp.int32),
            pltpu.VMEM((bpw, G4), jnp.float32),
            pltpu.SemaphoreType.DMA,
        ],
    )
    def k(m_hbm, idx_hbm, out_hbm, idx_v, rows_v, sem):
        wid = jax.lax.axis_index("s") * info.num_cores + jax.lax.axis_index("c")
        base = wid * bpw
        pltpu.sync_copy(idx_hbm.at[pl.ds(base, bpw)], idx_v)
        pltpu.async_copy(m_hbm.at[idx_v], rows_v, sem).wait()
        pltpu.sync_copy(rows_v, out_hbm.at[pl.ds(base, bpw)])

    return k(m, idx)


def _gates(z, c):
    i = jax.nn.sigmoid(z[:, 0 * U:1 * U])
    f = jax.nn.sigmoid(z[:, 1 * U:2 * U])
    g = jnp.tanh(z[:, 2 * U:3 * U])
    o = jax.nn.sigmoid(z[:, 3 * U:4 * U])
    c = f * c + i * g
    h = o * jnp.tanh(c)
    return h, c


def _rec_body(ze_ref, zd_ref, whe_ref, whd_ref, out_ref):
    whe = whe_ref[...]
    whd = whd_ref[...]

    def enc_step(t, carry):
        h, c = carry
        z = ze_ref[t] + jnp.dot(h, whe, preferred_element_type=jnp.float32)
        return _gates(z, c)

    zero = jnp.zeros((B, U), jnp.float32)
    h_e, c_e = jax.lax.fori_loop(0, S, enc_step, (zero, zero), unroll=4)

    def dec_step(t, carry):
        h, c = carry
        z = zd_ref[t] + jnp.dot(h, whd, preferred_element_type=jnp.float32)
        h, c = _gates(z, c)
        out_ref[t] = h
        return (h, c)

    jax.lax.fori_loop(0, S, dec_step, (h_e, c_e), unroll=4)


def _recurrence(z_e_t, z_d_t, Wh_e, Wh_d):
    return pl.pallas_call(
        _rec_body,
        out_shape=jax.ShapeDtypeStruct((S, B, U), jnp.float32),
    )(z_e_t, z_d_t, Wh_e, Wh_d)


def _head_body(x_ref, wd_ref, bd_ref, o_ref):
    logits = (
        jnp.dot(x_ref[...], wd_ref[...], preferred_element_type=jnp.float32)
        + bd_ref[...]
    )
    m = jnp.max(logits, axis=1, keepdims=True)
    e = jnp.exp(logits - m)
    o_ref[...] = e / jnp.sum(e, axis=1, keepdims=True)


def _softmax_head(x, Wd, bd):
    return pl.pallas_call(
        _head_body,
        out_shape=jax.ShapeDtypeStruct((BS, V), jnp.float32),
    )(x, Wd, bd.reshape(1, V))


def kernel(encoder_input, decoder_input, emb1, emb2, Wi_e, Wh_e, b_e,
           Wi_d, Wh_d, b_d, Wd, bd):
    idx_e = encoder_input.reshape(BS)
    idx_d = decoder_input.reshape(BS)
    m_e = _mproj(emb1.T, Wi_e, b_e)
    m_d = _mproj(emb2.T, Wi_d, b_d)
    z_e = _sc_gather_rows(m_e, idx_e)
    z_d = _sc_gather_rows(m_d, idx_d)
    z_e_t = z_e.reshape(B, S, G4).transpose(1, 0, 2)
    z_d_t = z_d.reshape(B, S, G4).transpose(1, 0, 2)
    dec_out = _recurrence(z_e_t, z_d_t, Wh_e, Wh_d)
    x = dec_out.transpose(1, 0, 2).reshape(BS, U)
    prbs = _softmax_head(x, Wd, bd)
    return prbs.reshape(B, S, V)


# speedup vs baseline: 1.7600x; 1.0220x over previous
import functools

import jax
import jax.numpy as jnp
from jax.experimental import pallas as pl
from jax.experimental.pallas import tpu as pltpu
from jax.experimental.pallas import tpu_sc as plsc

B, S = 8, 64
V, D, U = 2048, 22000, 64
BS = B * S
G4 = 4 * U
KT = 2048
NK = (D + KT - 1) // KT


def _mproj_body(et_ref, wi_ref, b_ref, m_ref):
    j = pl.program_id(0)

    @pl.when(j == 0)
    def _():
        m_ref[...] = jnp.broadcast_to(b_ref[...], (V, G4))

    row = j * KT + jax.lax.broadcasted_iota(jnp.int32, (KT, 1), 0)
    et = jnp.where(row < D, et_ref[...], 0.0)
    wi = jnp.where(row < D, wi_ref[...], 0.0)
    m_ref[...] += jax.lax.dot_general(
        et.astype(jnp.bfloat16), wi.astype(jnp.bfloat16),
        dimension_numbers=(((0,), (0,)), ((), ())),
        preferred_element_type=jnp.float32)


def _mproj(emb_t, Wi, b):
    return pl.pallas_call(
        _mproj_body,
        grid=(NK,),
        in_specs=[
            pl.BlockSpec((KT, V), lambda j: (j, 0)),
            pl.BlockSpec((KT, G4), lambda j: (j, 0)),
            pl.BlockSpec((1, G4), lambda j: (0, 0)),
        ],
        out_specs=pl.BlockSpec((V, G4), lambda j: (0, 0)),
        out_shape=jax.ShapeDtypeStruct((V, G4), jnp.float32),
    )(emb_t, Wi, b.reshape(1, G4))


def _sc_gather_rows(m, idx):
    info = plsc.get_sparse_core_info()
    nw = info.num_cores * info.num_subcores
    bpw = BS // nw
    mesh = plsc.VectorSubcoreMesh(core_axis_name="c", subcore_axis_name="s")

    @functools.partial(
        pl.kernel,
        mesh=mesh,
        out_type=jax.ShapeDtypeStruct((BS, G4), jnp.float32),
        scratch_types=[
            pltpu.VMEM((bpw,), jnp.int32),
            pltpu.VMEM((bpw, G4), jnp.float32),
            pltpu.SemaphoreType.DMA,
        ],
    )
    def k(m_hbm, idx_hbm, out_hbm, idx_v, rows_v, sem):
        wid = jax.lax.axis_index("s") * info.num_cores + jax.lax.axis_index("c")
        base = wid * bpw
        pltpu.sync_copy(idx_hbm.at[pl.ds(base, bpw)], idx_v)
        pltpu.async_copy(m_hbm.at[idx_v], rows_v, sem).wait()
        pltpu.sync_copy(rows_v, out_hbm.at[pl.ds(base, bpw)])

    return k(m, idx)


def _gates(z, c):
    i = jax.nn.sigmoid(z[:, 0 * U:1 * U])
    f = jax.nn.sigmoid(z[:, 1 * U:2 * U])
    g = jnp.tanh(z[:, 2 * U:3 * U])
    o = jax.nn.sigmoid(z[:, 3 * U:4 * U])
    c = f * c + i * g
    h = o * jnp.tanh(c)
    return h, c


def _rec_body(ze_ref, zd_ref, whe_ref, whd_ref, out_ref):
    whe = whe_ref[...]
    whd = whd_ref[...]

    def enc_step(t, carry):
        h, c = carry
        z = ze_ref[t] + jnp.dot(h, whe, preferred_element_type=jnp.float32)
        return _gates(z, c)

    zero = jnp.zeros((B, U), jnp.float32)
    h_e, c_e = jax.lax.fori_loop(0, S, enc_step, (zero, zero), unroll=4)

    def dec_step(t, carry):
        h, c = carry
        z = zd_ref[t] + jnp.dot(h, whd, preferred_element_type=jnp.float32)
        h, c = _gates(z, c)
        out_ref[:, t, :] = h
        return (h, c)

    jax.lax.fori_loop(0, S, dec_step, (h_e, c_e), unroll=4)


def _recurrence(z_e_t, z_d_t, Wh_e, Wh_d):
    return pl.pallas_call(
        _rec_body,
        out_shape=jax.ShapeDtypeStruct((B, S, U), jnp.float32),
    )(z_e_t, z_d_t, Wh_e, Wh_d)


def _head_body(x_ref, wd_ref, bd_ref, o_ref):
    logits = (
        jnp.dot(x_ref[...], wd_ref[...], preferred_element_type=jnp.float32)
        + bd_ref[...]
    )
    m = jnp.max(logits, axis=1, keepdims=True)
    e = jnp.exp(logits - m)
    o_ref[...] = e / jnp.sum(e, axis=1, keepdims=True)


def _softmax_head(x, Wd, bd):
    return pl.pallas_call(
        _head_body,
        out_shape=jax.ShapeDtypeStruct((BS, V), jnp.float32),
    )(x, Wd, bd.reshape(1, V))


def kernel(encoder_input, decoder_input, emb1, emb2, Wi_e, Wh_e, b_e,
           Wi_d, Wh_d, b_d, Wd, bd):
    idx_e = encoder_input.T.reshape(BS)
    idx_d = decoder_input.T.reshape(BS)
    m_e = _mproj(emb1.T, Wi_e, b_e)
    m_d = _mproj(emb2.T, Wi_d, b_d)
    z_e_t = _sc_gather_rows(m_e, idx_e).reshape(S, B, G4)
    z_d_t = _sc_gather_rows(m_d, idx_d).reshape(S, B, G4)
    dec_out = _recurrence(z_e_t, z_d_t, Wh_e, Wh_d)
    x = dec_out.reshape(BS, U)
    prbs = _softmax_head(x, Wd, bd)
    return prbs.reshape(B, S, V)
